# Initial kernel scaffold; baseline (speedup 1.0000x reference)
import jax
import jax.numpy as jnp
from jax.experimental import pallas as pl


def kernel(key_list, key_embed, query_list, query_embed, a, a_2, trans):
    raise NotImplementedError("write your pallas kernel here")


# speedup vs baseline: 3.3481x; 3.3481x over previous
import functools

import jax
import jax.numpy as jnp
from jax import lax
from jax.experimental import pallas as pl
from jax.experimental.pallas import tpu as pltpu
from jax.experimental.pallas import tpu_sc as plsc

EDIM = 128
ODIM = 32
PAD = 48
ECOL = 32
NQ = 10000
T_TOTAL = 320000
SLOPE = 0.2

BT = 2000
BQ = 2048

NC = 2
NS = 16
NW = NC * NS
PER_W = T_TOTAL // NW
CH = 1000
NCH = PER_W // CH
SUB = 8
SUBW = CH // SUB
NQPAD = 10240
ROWS_PER_TILE = NQPAD // NS


def _payload_body(key_ref, query_ref, a_ref, a2_ref, transp_ref, out_ref):
    key = key_ref[...]
    query = query_ref[...]
    av = jnp.dot(a2_ref[...], a_ref[...], preferred_element_type=jnp.float32)
    vk = av[:, :EDIM]
    vq = av[:, EDIM:]
    dn = (((1,), (1,)), ((), ()))
    assoc = lax.dot_general(key, transp_ref[...], dn,
                            preferred_element_type=jnp.float32)
    s = (lax.dot_general(key, vk, dn, preferred_element_type=jnp.float32) +
         lax.dot_general(query, vq, dn, preferred_element_type=jnp.float32))
    e = jnp.exp(-jnp.maximum(s, SLOPE * s))
    col = lax.broadcasted_iota(jnp.int32, (BT, PAD), 1)
    out_ref[...] = jnp.where(col == ECOL, e, assoc * e)


def _scatter_body(payload_hbm, qidx_hbm, zeros_hbm, out_hbm,
                  buf, idxbuf, acc, sem_p, sem_i):
    c = lax.axis_index("c")
    s = lax.axis_index("s")
    wid = c * NS + s

    pltpu.sync_copy(zeros_hbm.at[pl.ds(s * ROWS_PER_TILE, ROWS_PER_TILE)],
                    acc.at[pl.ds(s * ROWS_PER_TILE, ROWS_PER_TILE)])

    def start(i, slot):
        p = pltpu.async_copy(
            payload_hbm.at[pl.ds(wid * PER_W + i * CH, CH)], buf.at[slot], sem_p)
        q = pltpu.async_copy(qidx_hbm.at[wid * NCH + i], idxbuf.at[slot], sem_i)
        return p, q

    pending = start(0, 0)
    plsc.subcore_barrier()

    for i in range(NCH):
        slot = i % 2
        pending[0].wait()
        pending[1].wait()
        if i + 1 < NCH:
            nxt = start(i + 1, (i + 1) % 2)
        for j in range(SUB):
            pltpu.sync_copy(buf.at[slot, pl.ds(j * SUBW, SUBW)],
                            acc.at[idxbuf.at[slot, j]], add=True)
        if i + 1 < NCH:
            pending = nxt

    plsc.subcore_barrier()
    pltpu.sync_copy(acc.at[pl.ds(s * ROWS_PER_TILE, ROWS_PER_TILE)],
                    out_hbm.at[c, pl.ds(s * ROWS_PER_TILE, ROWS_PER_TILE)])


def _finalize_body(acc_ref, out_ref):
    p = acc_ref[0] + acc_ref[1]
    h = p[:, :ODIM]
    r = p[:, ECOL:ECOL + 1]
    r = jnp.where(r == 0.0, 1e-12, r)
    x = h / r
    out_ref[...] = jnp.where(x > 0.0, x, jnp.exp(x) - 1.0)


def kernel(key_list, key_embed, query_list, query_embed, a, a_2, trans):
    del key_list
    t = key_embed.shape[0]

    trans_pad = jnp.concatenate(
        [trans, jnp.zeros((PAD - ODIM, EDIM), trans.dtype)], axis=0)

    payload = pl.pallas_call(
        _payload_body,
        grid=(t // BT,),
        in_specs=[
            pl.BlockSpec((BT, EDIM), lambda i: (i, 0)),
            pl.BlockSpec((BT, EDIM), lambda i: (i, 0)),
            pl.BlockSpec((ODIM, 2 * EDIM), lambda i: (0, 0)),
            pl.BlockSpec((1, ODIM), lambda i: (0, 0)),
            pl.BlockSpec((PAD, EDIM), lambda i: (0, 0)),
        ],
        out_specs=pl.BlockSpec((BT, PAD), lambda i: (i, 0)),
        out_shape=jax.ShapeDtypeStruct((t, PAD), jnp.float32),
    )(key_embed, query_embed, a, a_2, trans_pad)

    qidx3 = query_list.astype(jnp.int32).reshape(t // CH, SUB, SUBW)
    zeros = jnp.zeros((NQPAD, PAD), jnp.float32)

    mesh = plsc.VectorSubcoreMesh(core_axis_name="c", subcore_axis_name="s")
    scatter = functools.partial(
        pl.kernel,
        out_type=jax.ShapeDtypeStruct((NC, NQPAD, PAD), jnp.float32),
        mesh=mesh,
        compiler_params=pltpu.CompilerParams(use_tc_tiling_on_sc=False),
        scratch_types=[
            pltpu.VMEM((2, CH, PAD), jnp.float32),
            pltpu.VMEM((2, SUB, SUBW), jnp.int32),
            pltpu.VMEM_SHARED((NQPAD, PAD), jnp.float32),
            pltpu.SemaphoreType.DMA,
            pltpu.SemaphoreType.DMA,
        ],
    )(_scatter_body)
    acc2 = scatter(payload, qidx3, zeros)

    out = pl.pallas_call(
        _finalize_body,
        grid=(NQPAD // BQ,),
        in_specs=[pl.BlockSpec((NC, BQ, PAD), lambda i: (0, i, 0))],
        out_specs=pl.BlockSpec((BQ, ODIM), lambda i: (i, 0)),
        out_shape=jax.ShapeDtypeStruct((NQPAD, ODIM), jnp.float32),
    )(acc2)
    return out[:NQ]


# speedup vs baseline: 3.6601x; 1.0932x over previous
import functools

import jax
import jax.numpy as jnp
from jax import lax
from jax.experimental import pallas as pl
from jax.experimental.pallas import tpu as pltpu
from jax.experimental.pallas import tpu_sc as plsc

EDIM = 128
ODIM = 32
PAD = 128
ECOL = 32
NQ = 10000
T_TOTAL = 320000
SLOPE = 0.2

BT = 2000
BQ = 2048

NC = 2
NS = 16
NW = NC * NS
PER_W = T_TOTAL // NW
CH = 100
NCH = PER_W // CH
SUB = 1
SUBW = CH // SUB
NQPAD = 10240
ROWS_PER_TILE = NQPAD // NS


def _payload_body(key_ref, query_ref, a_ref, a2_ref, transp_ref, out_ref):
    key = key_ref[...]
    query = query_ref[...]
    av = jnp.dot(a2_ref[...], a_ref[...], preferred_element_type=jnp.float32)
    vk = av[:, :EDIM]
    vq = av[:, EDIM:]
    dn = (((1,), (1,)), ((), ()))
    assoc = lax.dot_general(key, transp_ref[...], dn,
                            preferred_element_type=jnp.float32)
    s = (lax.dot_general(key, vk, dn, preferred_element_type=jnp.float32) +
         lax.dot_general(query, vq, dn, preferred_element_type=jnp.float32))
    e = jnp.exp(-jnp.maximum(s, SLOPE * s))
    col = lax.broadcasted_iota(jnp.int32, (BT, PAD), 1)
    out_ref[...] = jnp.where(col == ECOL, e, assoc * e)


def _scatter_body(payload_hbm, qidx_hbm, zeros_hbm, out_hbm,
                  buf, idxbuf, acc, sem_p, sem_i):
    c = lax.axis_index("c")
    s = lax.axis_index("s")
    wid = c * NS + s

    pltpu.sync_copy(zeros_hbm.at[pl.ds(s * ROWS_PER_TILE, ROWS_PER_TILE)],
                    acc.at[pl.ds(s * ROWS_PER_TILE, ROWS_PER_TILE)])

    def start(i, slot):
        p = pltpu.async_copy(
            payload_hbm.at[pl.ds(wid * PER_W + i * CH, CH)], buf.at[slot], sem_p)
        q = pltpu.async_copy(qidx_hbm.at[wid * NCH + i], idxbuf.at[slot], sem_i)
        return p, q

    pending = start(0, 0)
    plsc.subcore_barrier()

    for i in range(NCH):
        slot = i % 2
        pending[0].wait()
        pending[1].wait()
        if i + 1 < NCH:
            nxt = start(i + 1, (i + 1) % 2)
        for j in range(SUB):
            pltpu.sync_copy(buf.at[slot, pl.ds(j * SUBW, SUBW)],
                            acc.at[idxbuf.at[slot, j]], add=True)
        if i + 1 < NCH:
            pending = nxt

    plsc.subcore_barrier()
    pltpu.sync_copy(acc.at[pl.ds(s * ROWS_PER_TILE, ROWS_PER_TILE)],
                    out_hbm.at[c, pl.ds(s * ROWS_PER_TILE, ROWS_PER_TILE)])


def _finalize_body(acc_ref, out_ref):
    p = acc_ref[0] + acc_ref[1]
    h = p[:, :ODIM]
    r = p[:, ECOL:ECOL + 1]
    r = jnp.where(r == 0.0, 1e-12, r)
    x = h / r
    out_ref[...] = jnp.where(x > 0.0, x, jnp.exp(x) - 1.0)


def kernel(key_list, key_embed, query_list, query_embed, a, a_2, trans):
    del key_list
    t = key_embed.shape[0]

    trans_pad = jnp.concatenate(
        [trans, jnp.zeros((PAD - ODIM, EDIM), trans.dtype)], axis=0)

    payload = pl.pallas_call(
        _payload_body,
        grid=(t // BT,),
        in_specs=[
            pl.BlockSpec((BT, EDIM), lambda i: (i, 0)),
            pl.BlockSpec((BT, EDIM), lambda i: (i, 0)),
            pl.BlockSpec((ODIM, 2 * EDIM), lambda i: (0, 0)),
            pl.BlockSpec((1, ODIM), lambda i: (0, 0)),
            pl.BlockSpec((PAD, EDIM), lambda i: (0, 0)),
        ],
        out_specs=pl.BlockSpec((BT, PAD), lambda i: (i, 0)),
        out_shape=jax.ShapeDtypeStruct((t, PAD), jnp.float32),
    )(key_embed, query_embed, a, a_2, trans_pad)

    qidx3 = query_list.astype(jnp.int32).reshape(t // CH, SUB, SUBW)
    zeros = jnp.zeros((NQPAD, PAD), jnp.float32)

    mesh = plsc.VectorSubcoreMesh(core_axis_name="c", subcore_axis_name="s")
    scatter = functools.partial(
        pl.kernel,
        out_type=jax.ShapeDtypeStruct((NC, NQPAD, PAD), jnp.float32),
        mesh=mesh,
        compiler_params=pltpu.CompilerParams(use_tc_tiling_on_sc=False),
        scratch_types=[
            pltpu.VMEM((2, CH, PAD), jnp.float32),
            pltpu.VMEM((2, SUB, SUBW), jnp.int32),
            pltpu.VMEM_SHARED((NQPAD, PAD), jnp.float32),
            pltpu.SemaphoreType.DMA,
            pltpu.SemaphoreType.DMA,
        ],
    )(_scatter_body)
    acc2 = scatter(payload, qidx3, zeros)

    out = pl.pallas_call(
        _finalize_body,
        grid=(NQPAD // BQ,),
        in_specs=[pl.BlockSpec((NC, BQ, PAD), lambda i: (0, i, 0))],
        out_specs=pl.BlockSpec((BQ, ODIM), lambda i: (i, 0)),
        out_shape=jax.ShapeDtypeStruct((NQPAD, ODIM), jnp.float32),
    )(acc2)
    return out[:NQ]


# speedup vs baseline: 4.0973x; 1.1194x over previous
import functools

import jax
import jax.numpy as jnp
from jax import lax
from jax.experimental import pallas as pl
from jax.experimental.pallas import tpu as pltpu
from jax.experimental.pallas import tpu_sc as plsc

EDIM = 128
ODIM = 32
PAD = 128
ECOL = 32
NQ = 10000
T_TOTAL = 320000
SLOPE = 0.2

BT = 2000
BQ = 2048

NC = 2
NS = 16
NW = NC * NS
PER_W = T_TOTAL // NW
SCW = 48
CH = 200
NCH = PER_W // CH
SUB = 2
SUBW = CH // SUB
NQPAD = 10240
ROWS_PER_TILE = NQPAD // NS


def _payload_body(key_ref, query_ref, a_ref, a2_ref, transp_ref, out_ref):
    key = key_ref[...]
    query = query_ref[...]
    av = jnp.dot(a2_ref[...], a_ref[...], preferred_element_type=jnp.float32)
    vk = av[:, :EDIM]
    vq = av[:, EDIM:]
    dn = (((1,), (1,)), ((), ()))
    assoc = lax.dot_general(key, transp_ref[...], dn,
                            preferred_element_type=jnp.float32)
    s = (lax.dot_general(key, vk, dn, preferred_element_type=jnp.float32) +
         lax.dot_general(query, vq, dn, preferred_element_type=jnp.float32))
    e = jnp.exp(-jnp.maximum(s, SLOPE * s))
    col = lax.broadcasted_iota(jnp.int32, (BT, PAD), 1)
    out_ref[...] = jnp.where(col == ECOL, e, assoc * e)


def _scatter_body(payload_hbm, qidx_hbm, zeros_hbm, out_hbm,
                  buf, cbuf, idxbuf, acc, sem_p, sem_i):
    c = lax.axis_index("c")
    s = lax.axis_index("s")
    wid = c * NS + s
    base = wid * PER_W

    pltpu.sync_copy(zeros_hbm.at[pl.ds(s * ROWS_PER_TILE, ROWS_PER_TILE)],
                    acc.at[pl.ds(s * ROWS_PER_TILE, ROWS_PER_TILE)])

    def start(i, slot):
        pltpu.async_copy(payload_hbm.at[pl.ds(base + i * CH, CH)],
                         buf.at[slot], sem_p)
        pltpu.async_copy(qidx_hbm.at[wid * NCH + i], idxbuf.at[slot], sem_i)

    def wait(slot):
        pltpu.make_async_copy(payload_hbm.at[pl.ds(base, CH)],
                              buf.at[slot], sem_p).wait()
        pltpu.make_async_copy(qidx_hbm.at[wid * NCH], idxbuf.at[slot], sem_i).wait()

    def consume(slot):
        for j in range(CH):
            for g in range(SCW // 16):
                cbuf[slot, j, pl.ds(g * 16, 16)] = buf[slot, j, pl.ds(g * 16, 16)]
        for jj in range(SUB):
            pltpu.sync_copy(cbuf.at[slot, pl.ds(jj * SUBW, SUBW)],
                            acc.at[idxbuf.at[slot, jj]], add=True)

    start(0, 0)
    plsc.subcore_barrier()

    def body(k, carry):
        i0 = k * 2
        wait(0)
        start(i0 + 1, 1)
        consume(0)
        wait(1)

        @pl.when(i0 + 2 < NCH)
        def _():
            start(i0 + 2, 0)

        consume(1)
        return carry

    lax.fori_loop(0, NCH // 2, body, 0)

    plsc.subcore_barrier()
    pltpu.sync_copy(acc.at[pl.ds(s * ROWS_PER_TILE, ROWS_PER_TILE)],
                    out_hbm.at[c, pl.ds(s * ROWS_PER_TILE, ROWS_PER_TILE)])


def _finalize_body(acc_ref, out_ref):
    p = acc_ref[0] + acc_ref[1]
    h = p[:, :ODIM]
    r = p[:, ECOL:ECOL + 1]
    r = jnp.where(r == 0.0, 1e-12, r)
    x = h / r
    out_ref[...] = jnp.where(x > 0.0, x, jnp.exp(x) - 1.0)


def kernel(key_list, key_embed, query_list, query_embed, a, a_2, trans):
    del key_list
    t = key_embed.shape[0]

    trans_pad = jnp.concatenate(
        [trans, jnp.zeros((PAD - ODIM, EDIM), trans.dtype)], axis=0)

    payload = pl.pallas_call(
        _payload_body,
        grid=(t // BT,),
        in_specs=[
            pl.BlockSpec((BT, EDIM), lambda i: (i, 0)),
            pl.BlockSpec((BT, EDIM), lambda i: (i, 0)),
            pl.BlockSpec((ODIM, 2 * EDIM), lambda i: (0, 0)),
            pl.BlockSpec((1, ODIM), lambda i: (0, 0)),
            pl.BlockSpec((PAD, EDIM), lambda i: (0, 0)),
        ],
        out_specs=pl.BlockSpec((BT, PAD), lambda i: (i, 0)),
        out_shape=jax.ShapeDtypeStruct((t, PAD), jnp.float32),
    )(key_embed, query_embed, a, a_2, trans_pad)

    qidx3 = query_list.astype(jnp.int32).reshape(t // CH, SUB, SUBW)
    zeros = jnp.zeros((NQPAD, SCW), jnp.float32)

    mesh = plsc.VectorSubcoreMesh(core_axis_name="c", subcore_axis_name="s")
    scatter = functools.partial(
        pl.kernel,
        out_type=jax.ShapeDtypeStruct((NC, NQPAD, SCW), jnp.float32),
        mesh=mesh,
        compiler_params=pltpu.CompilerParams(use_tc_tiling_on_sc=False),
        scratch_types=[
            pltpu.VMEM((2, CH, PAD), jnp.float32),
            pltpu.VMEM((2, CH, SCW), jnp.float32),
            pltpu.VMEM((2, SUB, SUBW), jnp.int32),
            pltpu.VMEM_SHARED((NQPAD, SCW), jnp.float32),
            pltpu.SemaphoreType.DMA,
            pltpu.SemaphoreType.DMA,
        ],
    )(_scatter_body)
    acc2 = scatter(payload, qidx3, zeros)

    out = pl.pallas_call(
        _finalize_body,
        grid=(NQPAD // BQ,),
        in_specs=[pl.BlockSpec((NC, BQ, SCW), lambda i: (0, i, 0))],
        out_specs=pl.BlockSpec((BQ, ODIM), lambda i: (i, 0)),
        out_shape=jax.ShapeDtypeStruct((NQPAD, ODIM), jnp.float32),
    )(acc2)
    return out[:NQ]


# speedup vs baseline: 4.1992x; 1.0249x over previous
import functools

import jax
import jax.numpy as jnp
from jax import lax
from jax.experimental import pallas as pl
from jax.experimental.pallas import tpu as pltpu
from jax.experimental.pallas import tpu_sc as plsc

EDIM = 128
ODIM = 32
PAD = 128
ECOL = 32
NQ = 10000
T_TOTAL = 320000
SLOPE = 0.2

BT = 2000
BQ = 2048

NC = 2
NS = 16
NW = NC * NS
NSPLIT = 2
HALF = T_TOTAL // NSPLIT
PER_W = HALF // NW
SCW = 48
CH = 100
NCH = PER_W // CH
SUB = 1
SUBW = CH // SUB
NQPAD = 10240
ROWS_PER_TILE = NQPAD // NS


def _payload_body(key_ref, query_ref, a_ref, a2_ref, transp_ref, out_ref):
    key = key_ref[...]
    query = query_ref[...]
    av = jnp.dot(a2_ref[...], a_ref[...], preferred_element_type=jnp.float32)
    vk = av[:, :EDIM]
    vq = av[:, EDIM:]
    dn = (((1,), (1,)), ((), ()))
    assoc = lax.dot_general(key, transp_ref[...], dn,
                            preferred_element_type=jnp.float32)
    s = (lax.dot_general(key, vk, dn, preferred_element_type=jnp.float32) +
         lax.dot_general(query, vq, dn, preferred_element_type=jnp.float32))
    e = jnp.exp(-jnp.maximum(s, SLOPE * s))
    col = lax.broadcasted_iota(jnp.int32, (BT, PAD), 1)
    out_ref[...] = jnp.where(col == ECOL, e, assoc * e)


def _scatter_body(payload_hbm, qidx_hbm, zeros_hbm, out_hbm,
                  buf, cbuf, idxbuf, acc, sem_p, sem_i):
    c = lax.axis_index("c")
    s = lax.axis_index("s")
    wid = c * NS + s
    base = wid * PER_W

    pltpu.sync_copy(zeros_hbm.at[pl.ds(s * ROWS_PER_TILE, ROWS_PER_TILE)],
                    acc.at[pl.ds(s * ROWS_PER_TILE, ROWS_PER_TILE)])

    def start(i, slot):
        pltpu.async_copy(payload_hbm.at[pl.ds(base + i * CH, CH)],
                         buf.at[slot], sem_p)
        pltpu.async_copy(qidx_hbm.at[wid * NCH + i], idxbuf.at[slot], sem_i)

    def wait(slot):
        pltpu.make_async_copy(payload_hbm.at[pl.ds(base, CH)],
                              buf.at[slot], sem_p).wait()
        pltpu.make_async_copy(qidx_hbm.at[wid * NCH], idxbuf.at[slot], sem_i).wait()

    def consume(slot):
        for j in range(CH):
            for g in range(SCW // 16):
                cbuf[slot, j, pl.ds(g * 16, 16)] = buf[slot, j, pl.ds(g * 16, 16)]
        for jj in range(SUB):
            pltpu.sync_copy(cbuf.at[slot, pl.ds(jj * SUBW, SUBW)],
                            acc.at[idxbuf.at[slot, jj]], add=True)

    start(0, 0)
    plsc.subcore_barrier()

    def body(k, carry):
        i0 = k * 2
        wait(0)
        start(i0 + 1, 1)
        consume(0)
        wait(1)

        @pl.when(i0 + 2 < NCH)
        def _():
            start(i0 + 2, 0)

        consume(1)
        return carry

    lax.fori_loop(0, NCH // 2, body, 0)

    plsc.subcore_barrier()
    pltpu.sync_copy(acc.at[pl.ds(s * ROWS_PER_TILE, ROWS_PER_TILE)],
                    out_hbm.at[c, pl.ds(s * ROWS_PER_TILE, ROWS_PER_TILE)])


def _finalize_body(acc_a_ref, acc_b_ref, out_ref):
    p = (acc_a_ref[0] + acc_a_ref[1]) + (acc_b_ref[0] + acc_b_ref[1])
    h = p[:, :ODIM]
    r = p[:, ECOL:ECOL + 1]
    r = jnp.where(r == 0.0, 1e-12, r)
    x = h / r
    out_ref[...] = jnp.where(x > 0.0, x, jnp.exp(x) - 1.0)


def kernel(key_list, key_embed, query_list, query_embed, a, a_2, trans):
    del key_list
    t = key_embed.shape[0]

    trans_pad = jnp.concatenate(
        [trans, jnp.zeros((PAD - ODIM, EDIM), trans.dtype)], axis=0)

    def payload_half(half):
        off = half * (HALF // BT)
        return pl.pallas_call(
            _payload_body,
            grid=(HALF // BT,),
            in_specs=[
                pl.BlockSpec((BT, EDIM), lambda i: (i + off, 0)),
                pl.BlockSpec((BT, EDIM), lambda i: (i + off, 0)),
                pl.BlockSpec((ODIM, 2 * EDIM), lambda i: (0, 0)),
                pl.BlockSpec((1, ODIM), lambda i: (0, 0)),
                pl.BlockSpec((PAD, EDIM), lambda i: (0, 0)),
            ],
            out_specs=pl.BlockSpec((BT, PAD), lambda i: (i, 0)),
            out_shape=jax.ShapeDtypeStruct((HALF, PAD), jnp.float32),
        )(key_embed, query_embed, a, a_2, trans_pad)

    payloads = [payload_half(h) for h in range(NSPLIT)]

    qidx = query_list.astype(jnp.int32)
    zeros = jnp.zeros((NQPAD, SCW), jnp.float32)

    mesh = plsc.VectorSubcoreMesh(core_axis_name="c", subcore_axis_name="s")
    scatter = functools.partial(
        pl.kernel,
        out_type=jax.ShapeDtypeStruct((NC, NQPAD, SCW), jnp.float32),
        mesh=mesh,
        compiler_params=pltpu.CompilerParams(use_tc_tiling_on_sc=False),
        scratch_types=[
            pltpu.VMEM((2, CH, PAD), jnp.float32),
            pltpu.VMEM((2, CH, SCW), jnp.float32),
            pltpu.VMEM((2, SUB, SUBW), jnp.int32),
            pltpu.VMEM_SHARED((NQPAD, SCW), jnp.float32),
            pltpu.SemaphoreType.DMA,
            pltpu.SemaphoreType.DMA,
        ],
    )(_scatter_body)

    accs = [
        scatter(payloads[h],
                qidx[h * HALF:(h + 1) * HALF].reshape(HALF // CH, SUB, SUBW),
                zeros)
        for h in range(NSPLIT)
    ]

    out = pl.pallas_call(
        _finalize_body,
        grid=(NQPAD // BQ,),
        in_specs=[pl.BlockSpec((NC, BQ, SCW), lambda i: (0, i, 0))] * NSPLIT,
        out_specs=pl.BlockSpec((BQ, ODIM), lambda i: (i, 0)),
        out_shape=jax.ShapeDtypeStruct((NQPAD, ODIM), jnp.float32),
    )(*accs)
    return out[:NQ]


# speedup vs baseline: 4.2000x; 1.0002x over previous
import functools

import jax
import jax.numpy as jnp
from jax import lax
from jax.experimental import pallas as pl
from jax.experimental.pallas import tpu as pltpu
from jax.experimental.pallas import tpu_sc as plsc

EDIM = 128
ODIM = 32
PAD = 128
ECOL = 32
NQ = 10000
T_TOTAL = 320000
SLOPE = 0.2

BT = 2000
BQ = 2048

NC = 2
NS = 16
NW = NC * NS
PER_W = T_TOTAL // NW
SCW = 48
CH = 400
CHP = CH // 2
NCH = PER_W // CH
SUB = 4
SUBW = CH // SUB
NQPAD = 10240
ROWS_PER_TILE = NQPAD // NS


def _payload_body(key_ref, query_ref, a_ref, a2_ref, transp_ref, out_ref):
    key = key_ref[...]
    query = query_ref[...]
    av = jnp.dot(a2_ref[...], a_ref[...], preferred_element_type=jnp.float32)
    vk = av[:, :EDIM]
    vq = av[:, EDIM:]
    dn = (((1,), (1,)), ((), ()))
    assoc = lax.dot_general(key, transp_ref[...], dn,
                            preferred_element_type=jnp.float32)
    s = (lax.dot_general(key, vk, dn, preferred_element_type=jnp.float32) +
         lax.dot_general(query, vq, dn, preferred_element_type=jnp.float32))
    e = jnp.exp(-jnp.maximum(s, SLOPE * s))
    col = lax.broadcasted_iota(jnp.int32, (BT, PAD), 1)
    pay = jnp.where(col == ECOL, e, assoc * e)
    out_ref[...] = pltpu.bitcast(pay.astype(jnp.bfloat16), jnp.int32)


def _scatter_body(payload_hbm, qidx_hbm, zeros_hbm, out_hbm,
                  buf, cbuf, idxbuf, acc, sem_p, sem_i):
    c = lax.axis_index("c")
    s = lax.axis_index("s")
    wid = c * NS + s
    base = wid * (PER_W // 2)

    pltpu.sync_copy(zeros_hbm.at[pl.ds(s * ROWS_PER_TILE, ROWS_PER_TILE)],
                    acc.at[pl.ds(s * ROWS_PER_TILE, ROWS_PER_TILE)])

    def start(i, slot):
        pltpu.async_copy(payload_hbm.at[pl.ds(base + i * CHP, CHP)],
                         buf.at[slot], sem_p)
        pltpu.async_copy(qidx_hbm.at[wid * NCH + i], idxbuf.at[slot], sem_i)

    def wait(slot):
        pltpu.make_async_copy(payload_hbm.at[pl.ds(base, CHP)],
                              buf.at[slot], sem_p).wait()
        pltpu.make_async_copy(qidx_hbm.at[wid * NCH], idxbuf.at[slot], sem_i).wait()

    def consume(slot):
        def unpack_rows(r, carry):
            for u in range(2):
                rr = r * 2 + u
                for g in range(SCW // 16):
                    w = buf[slot, rr, pl.ds(g * 16, 16)]
                    lo = lax.bitcast_convert_type(w << 16, jnp.float32)
                    hi = lax.bitcast_convert_type((w >> 16) << 16, jnp.float32)
                    cbuf[slot, 2 * rr, pl.ds(g * 16, 16)] = lo
                    cbuf[slot, 2 * rr + 1, pl.ds(g * 16, 16)] = hi
            return carry

        lax.fori_loop(0, CHP // 2, unpack_rows, 0)
        for jj in range(SUB):
            pltpu.sync_copy(cbuf.at[slot, pl.ds(jj * SUBW, SUBW)],
                            acc.at[idxbuf.at[slot, jj]], add=True)

    start(0, 0)
    plsc.subcore_barrier()

    def body(k, carry):
        i0 = k * 2
        wait(0)
        start(i0 + 1, 1)
        consume(0)
        wait(1)

        @pl.when(i0 + 2 < NCH)
        def _():
            start(i0 + 2, 0)

        consume(1)
        return carry

    lax.fori_loop(0, (NCH - 1) // 2, body, 0)
    wait(0)
    consume(0)

    plsc.subcore_barrier()
    pltpu.sync_copy(acc.at[pl.ds(s * ROWS_PER_TILE, ROWS_PER_TILE)],
                    out_hbm.at[c, pl.ds(s * ROWS_PER_TILE, ROWS_PER_TILE)])


def _finalize_body(acc_ref, out_ref):
    p = acc_ref[0] + acc_ref[1]
    h = p[:, :ODIM]
    r = p[:, ECOL:ECOL + 1]
    r = jnp.where(r == 0.0, 1e-12, r)
    x = h / r
    out_ref[...] = jnp.where(x > 0.0, x, jnp.exp(x) - 1.0)


def kernel(key_list, key_embed, query_list, query_embed, a, a_2, trans):
    del key_list
    t = key_embed.shape[0]

    trans_pad = jnp.concatenate(
        [trans, jnp.zeros((PAD - ODIM, EDIM), trans.dtype)], axis=0)

    payload = pl.pallas_call(
        _payload_body,
        grid=(t // BT,),
        in_specs=[
            pl.BlockSpec((BT, EDIM), lambda i: (i, 0)),
            pl.BlockSpec((BT, EDIM), lambda i: (i, 0)),
            pl.BlockSpec((ODIM, 2 * EDIM), lambda i: (0, 0)),
            pl.BlockSpec((1, ODIM), lambda i: (0, 0)),
            pl.BlockSpec((PAD, EDIM), lambda i: (0, 0)),
        ],
        out_specs=pl.BlockSpec((BT // 2, PAD), lambda i: (i, 0)),
        out_shape=jax.ShapeDtypeStruct((t // 2, PAD), jnp.int32),
    )(key_embed, query_embed, a, a_2, trans_pad)

    qidx3 = query_list.astype(jnp.int32).reshape(t // CH, SUB, SUBW)
    zeros = jnp.zeros((NQPAD, SCW), jnp.float32)

    mesh = plsc.VectorSubcoreMesh(core_axis_name="c", subcore_axis_name="s")
    scatter = functools.partial(
        pl.kernel,
        out_type=jax.ShapeDtypeStruct((NC, NQPAD, SCW), jnp.float32),
        mesh=mesh,
        compiler_params=pltpu.CompilerParams(use_tc_tiling_on_sc=False),
        scratch_types=[
            pltpu.VMEM((2, CHP, PAD), jnp.int32),
            pltpu.VMEM((2, CH, SCW), jnp.float32),
            pltpu.VMEM((2, SUB, SUBW), jnp.int32),
            pltpu.VMEM_SHARED((NQPAD, SCW), jnp.float32),
            pltpu.SemaphoreType.DMA,
            pltpu.SemaphoreType.DMA,
        ],
    )(_scatter_body)
    acc2 = scatter(payload, qidx3, zeros)

    out = pl.pallas_call(
        _finalize_body,
        grid=(NQPAD // BQ,),
        in_specs=[pl.BlockSpec((NC, BQ, SCW), lambda i: (0, i, 0))],
        out_specs=pl.BlockSpec((BQ, ODIM), lambda i: (i, 0)),
        out_shape=jax.ShapeDtypeStruct((NQPAD, ODIM), jnp.float32),
    )(acc2)
    return out[:NQ]


# speedup vs baseline: 4.3450x; 1.0345x over previous
import functools

import jax
import jax.numpy as jnp
from jax import lax
from jax.experimental import pallas as pl
from jax.experimental.pallas import tpu as pltpu
from jax.experimental.pallas import tpu_sc as plsc

EDIM = 128
ODIM = 32
PAD = 128
ECOL = 32
NQ = 10000
T_TOTAL = 320000
SLOPE = 0.2

BT = 2000
BQ = 2048

NC = 2
NS = 16
NW = NC * NS
PER_W = T_TOTAL // NW
SCW = 48
CH = 400
CHP = CH // 2
NCH = PER_W // CH
SUB = 4
SUBW = CH // SUB
NQPAD = 10240
ROWS_PER_TILE = NQPAD // NS


def _payload_body(key_ref, query_ref, a_ref, a2_ref, transp_ref, out_ref):
    key = key_ref[...]
    query = query_ref[...]
    av = jnp.dot(a2_ref[...], a_ref[...], preferred_element_type=jnp.float32)
    vk = av[:, :EDIM]
    vq = av[:, EDIM:]
    dn = (((1,), (1,)), ((), ()))
    assoc = lax.dot_general(key, transp_ref[...], dn,
                            preferred_element_type=jnp.float32)
    s = (lax.dot_general(key, vk, dn, preferred_element_type=jnp.float32) +
         lax.dot_general(query, vq, dn, preferred_element_type=jnp.float32))
    e = jnp.exp(-jnp.maximum(s, SLOPE * s))
    col = lax.broadcasted_iota(jnp.int32, (BT, PAD), 1)
    pay = jnp.where(col == ECOL, e, assoc * e)
    out_ref[...] = pltpu.bitcast(pay.astype(jnp.bfloat16), jnp.int32)


def _scatter_body(payload_hbm, qidx_hbm, zeros_hbm, out_hbm,
                  buf, cbuf, idxbuf, acc, sem_p, sem_i, sem_s):
    c = lax.axis_index("c")
    s = lax.axis_index("s")
    wid = c * NS + s
    base = wid * (PER_W // 2)

    pltpu.sync_copy(zeros_hbm.at[pl.ds(s * ROWS_PER_TILE, ROWS_PER_TILE)],
                    acc.at[pl.ds(s * ROWS_PER_TILE, ROWS_PER_TILE)])

    def start(i):
        bslot = lax.rem(i, 2)
        islot = lax.rem(i, 4)
        pltpu.async_copy(payload_hbm.at[pl.ds(base + i * CHP, CHP)],
                         buf.at[bslot], sem_p)
        pltpu.async_copy(qidx_hbm.at[wid * NCH + i], idxbuf.at[islot], sem_i)

    def waitload(i):
        bslot = lax.rem(i, 2)
        islot = lax.rem(i, 4)
        pltpu.make_async_copy(payload_hbm.at[pl.ds(base, CHP)],
                              buf.at[bslot], sem_p).wait()
        pltpu.make_async_copy(qidx_hbm.at[wid * NCH], idxbuf.at[islot], sem_i).wait()

    def unpack(i):
        bslot = lax.rem(i, 2)

        def unpack_rows(r, carry):
            for u in range(2):
                rr = r * 2 + u
                for g in range(SCW // 16):
                    w = buf[bslot, rr, pl.ds(g * 16, 16)]
                    lo = lax.bitcast_convert_type(w << 16, jnp.float32)
                    hi = lax.bitcast_convert_type((w >> 16) << 16, jnp.float32)
                    cbuf[bslot, 2 * rr, pl.ds(g * 16, 16)] = lo
                    cbuf[bslot, 2 * rr + 1, pl.ds(g * 16, 16)] = hi
            return carry

        lax.fori_loop(0, CHP // 2, unpack_rows, 0)

    def fire(i):
        cslot = lax.rem(i, 2)
        islot = lax.rem(i, 4)
        for jj in range(SUB):
            pltpu.async_copy(cbuf.at[cslot, pl.ds(jj * SUBW, SUBW)],
                             acc.at[idxbuf.at[islot, jj]], sem_s, add=True)

    def drain(i):
        cslot = lax.rem(i, 2)
        islot = lax.rem(i, 4)
        for jj in range(SUB):
            pltpu.make_async_copy(cbuf.at[cslot, pl.ds(jj * SUBW, SUBW)],
                                  acc.at[idxbuf.at[islot, jj]], sem_s).wait()

    start(0)
    start(1)
    plsc.subcore_barrier()

    def body(i, carry):
        waitload(i)

        @pl.when(i >= 2)
        def _():
            drain(i - 2)

        @pl.when(i + 2 < NCH)
        def _():
            start(i + 2)

        unpack(i)
        fire(i)
        return carry

    lax.fori_loop(0, NCH, body, 0)
    drain(NCH - 2)
    drain(NCH - 1)

    plsc.subcore_barrier()
    pltpu.sync_copy(acc.at[pl.ds(s * ROWS_PER_TILE, ROWS_PER_TILE)],
                    out_hbm.at[c, pl.ds(s * ROWS_PER_TILE, ROWS_PER_TILE)])


def _finalize_body(acc_ref, out_ref):
    p = acc_ref[0] + acc_ref[1]
    h = p[:, :ODIM]
    r = p[:, ECOL:ECOL + 1]
    r = jnp.where(r == 0.0, 1e-12, r)
    x = h / r
    out_ref[...] = jnp.where(x > 0.0, x, jnp.exp(x) - 1.0)


def kernel(key_list, key_embed, query_list, query_embed, a, a_2, trans):
    del key_list
    t = key_embed.shape[0]

    trans_pad = jnp.concatenate(
        [trans, jnp.zeros((PAD - ODIM, EDIM), trans.dtype)], axis=0)

    payload = pl.pallas_call(
        _payload_body,
        grid=(t // BT,),
        in_specs=[
            pl.BlockSpec((BT, EDIM), lambda i: (i, 0)),
            pl.BlockSpec((BT, EDIM), lambda i: (i, 0)),
            pl.BlockSpec((ODIM, 2 * EDIM), lambda i: (0, 0)),
            pl.BlockSpec((1, ODIM), lambda i: (0, 0)),
            pl.BlockSpec((PAD, EDIM), lambda i: (0, 0)),
        ],
        out_specs=pl.BlockSpec((BT // 2, PAD), lambda i: (i, 0)),
        out_shape=jax.ShapeDtypeStruct((t // 2, PAD), jnp.int32),
    )(key_embed, query_embed, a, a_2, trans_pad)

    qidx3 = query_list.astype(jnp.int32).reshape(t // CH, SUB, SUBW)
    zeros = jnp.zeros((NQPAD, SCW), jnp.float32)

    mesh = plsc.VectorSubcoreMesh(core_axis_name="c", subcore_axis_name="s")
    scatter = functools.partial(
        pl.kernel,
        out_type=jax.ShapeDtypeStruct((NC, NQPAD, SCW), jnp.float32),
        mesh=mesh,
        compiler_params=pltpu.CompilerParams(use_tc_tiling_on_sc=False),
        scratch_types=[
            pltpu.VMEM((2, CHP, PAD), jnp.int32),
            pltpu.VMEM((2, CH, SCW), jnp.float32),
            pltpu.VMEM((4, SUB, SUBW), jnp.int32),
            pltpu.VMEM_SHARED((NQPAD, SCW), jnp.float32),
            pltpu.SemaphoreType.DMA,
            pltpu.SemaphoreType.DMA,
            pltpu.SemaphoreType.DMA,
        ],
    )(_scatter_body)
    acc2 = scatter(payload, qidx3, zeros)

    out = pl.pallas_call(
        _finalize_body,
        grid=(NQPAD // BQ,),
        in_specs=[pl.BlockSpec((NC, BQ, SCW), lambda i: (0, i, 0))],
        out_specs=pl.BlockSpec((BQ, ODIM), lambda i: (i, 0)),
        out_shape=jax.ShapeDtypeStruct((NQPAD, ODIM), jnp.float32),
    )(acc2)
    return out[:NQ]


# speedup vs baseline: 4.3551x; 1.0023x over previous
import functools

import jax
import jax.numpy as jnp
from jax import lax
from jax.experimental import pallas as pl
from jax.experimental.pallas import tpu as pltpu
from jax.experimental.pallas import tpu_sc as plsc

EDIM = 128
ODIM = 32
PAD = 128
ECOL = 32
NQ = 10000
T_TOTAL = 320000
SLOPE = 0.2

BT = 2000
BQ = 2048

NC = 2
NS = 16
NW = NC * NS
PER_W = T_TOTAL // NW
SCW = 48
CH = 400
CHP = CH // 2
NCH = PER_W // CH
SUB = 4
SUBW = CH // SUB
NQPAD = 10240
ROWS_PER_TILE = NQPAD // NS


def _payload_body(key_ref, query_ref, a_ref, a2_ref, transp_ref, out_ref):
    key = key_ref[...]
    query = query_ref[...]
    av = jnp.dot(a2_ref[...], a_ref[...], preferred_element_type=jnp.float32)
    vk = av[:, :EDIM]
    vq = av[:, EDIM:]
    dn = (((1,), (1,)), ((), ()))
    assoc = lax.dot_general(key, transp_ref[...], dn,
                            preferred_element_type=jnp.float32)
    s = (lax.dot_general(key, vk, dn, preferred_element_type=jnp.float32) +
         lax.dot_general(query, vq, dn, preferred_element_type=jnp.float32))
    e = jnp.exp(-jnp.maximum(s, SLOPE * s))
    col = lax.broadcasted_iota(jnp.int32, (BT, PAD), 1)
    pay = jnp.where(col == ECOL, e, assoc * e)
    out_ref[...] = pltpu.bitcast(pay.astype(jnp.bfloat16), jnp.int32)


def _scatter_body(payload_hbm, qidx_hbm, zeros_hbm, out_hbm,
                  buf, cbuf, idxbuf, acc, sem_p, sem_i, sem_s):
    c = lax.axis_index("c")
    s = lax.axis_index("s")
    wid = c * NS + s
    base = wid * (PER_W // 2)

    pltpu.sync_copy(zeros_hbm.at[pl.ds(s * ROWS_PER_TILE, ROWS_PER_TILE)],
                    acc.at[pl.ds(s * ROWS_PER_TILE, ROWS_PER_TILE)])

    def start(i):
        bslot = lax.rem(i, 2)
        islot = lax.rem(i, 4)
        pltpu.async_copy(payload_hbm.at[pl.ds(base + i * CHP, CHP)],
                         buf.at[bslot], sem_p)
        pltpu.async_copy(qidx_hbm.at[wid * NCH + i], idxbuf.at[islot], sem_i)

    def waitload(i):
        bslot = lax.rem(i, 2)
        islot = lax.rem(i, 4)
        pltpu.make_async_copy(payload_hbm.at[pl.ds(base, CHP)],
                              buf.at[bslot], sem_p).wait()
        pltpu.make_async_copy(qidx_hbm.at[wid * NCH], idxbuf.at[islot], sem_i).wait()

    def unpack(i):
        bslot = lax.rem(i, 2)

        def unpack_rows(r, carry):
            for u in range(2):
                rr = r * 2 + u
                for g in range(SCW // 16):
                    w = buf[bslot, rr, pl.ds(g * 16, 16)]
                    lo = lax.bitcast_convert_type(w << 16, jnp.float32)
                    hi = lax.bitcast_convert_type((w >> 16) << 16, jnp.float32)
                    cbuf[bslot, 2 * rr, pl.ds(g * 16, 16)] = lo
                    cbuf[bslot, 2 * rr + 1, pl.ds(g * 16, 16)] = hi
            return carry

        lax.fori_loop(0, CHP // 2, unpack_rows, 0)

    def fire(i):
        cslot = lax.rem(i, 2)
        islot = lax.rem(i, 4)
        for jj in range(SUB):
            pltpu.async_copy(cbuf.at[cslot, pl.ds(jj * SUBW, SUBW)],
                             acc.at[idxbuf.at[islot, jj]], sem_s, add=True)

    def drain(i):
        cslot = lax.rem(i, 2)
        islot = lax.rem(i, 4)
        for jj in range(SUB):
            pltpu.make_async_copy(cbuf.at[cslot, pl.ds(jj * SUBW, SUBW)],
                                  acc.at[idxbuf.at[islot, jj]], sem_s).wait()

    start(0)
    start(1)
    plsc.subcore_barrier()

    def body(i, carry):
        waitload(i)

        @pl.when(i >= 2)
        def _():
            drain(i - 2)

        unpack(i)

        @pl.when(i + 2 < NCH)
        def _():
            start(i + 2)

        fire(i)
        return carry

    lax.fori_loop(0, NCH, body, 0)
    drain(NCH - 2)
    drain(NCH - 1)

    plsc.subcore_barrier()
    pltpu.sync_copy(acc.at[pl.ds(s * ROWS_PER_TILE, ROWS_PER_TILE)],
                    out_hbm.at[c, pl.ds(s * ROWS_PER_TILE, ROWS_PER_TILE)])


def _finalize_body(acc_ref, out_ref):
    p = acc_ref[0] + acc_ref[1]
    h = p[:, :ODIM]
    r = p[:, ECOL:ECOL + 1]
    r = jnp.where(r == 0.0, 1e-12, r)
    x = h / r
    out_ref[...] = jnp.where(x > 0.0, x, jnp.exp(x) - 1.0)


def kernel(key_list, key_embed, query_list, query_embed, a, a_2, trans):
    del key_list
    t = key_embed.shape[0]

    trans_pad = jnp.concatenate(
        [trans, jnp.zeros((PAD - ODIM, EDIM), trans.dtype)], axis=0)

    payload = pl.pallas_call(
        _payload_body,
        grid=(t // BT,),
        in_specs=[
            pl.BlockSpec((BT, EDIM), lambda i: (i, 0)),
            pl.BlockSpec((BT, EDIM), lambda i: (i, 0)),
            pl.BlockSpec((ODIM, 2 * EDIM), lambda i: (0, 0)),
            pl.BlockSpec((1, ODIM), lambda i: (0, 0)),
            pl.BlockSpec((PAD, EDIM), lambda i: (0, 0)),
        ],
        out_specs=pl.BlockSpec((BT // 2, PAD), lambda i: (i, 0)),
        out_shape=jax.ShapeDtypeStruct((t // 2, PAD), jnp.int32),
    )(key_embed, query_embed, a, a_2, trans_pad)

    qidx3 = query_list.astype(jnp.int32).reshape(t // CH, SUB, SUBW)
    zeros = jnp.zeros((NQPAD, SCW), jnp.float32)

    mesh = plsc.VectorSubcoreMesh(core_axis_name="c", subcore_axis_name="s")
    scatter = functools.partial(
        pl.kernel,
        out_type=jax.ShapeDtypeStruct((NC, NQPAD, SCW), jnp.float32),
        mesh=mesh,
        compiler_params=pltpu.CompilerParams(use_tc_tiling_on_sc=False),
        scratch_types=[
            pltpu.VMEM((2, CHP, PAD), jnp.int32),
            pltpu.VMEM((2, CH, SCW), jnp.float32),
            pltpu.VMEM((4, SUB, SUBW), jnp.int32),
            pltpu.VMEM_SHARED((NQPAD, SCW), jnp.float32),
            pltpu.SemaphoreType.DMA,
            pltpu.SemaphoreType.DMA,
            pltpu.SemaphoreType.DMA,
        ],
    )(_scatter_body)
    acc2 = scatter(payload, qidx3, zeros)

    out = pl.pallas_call(
        _finalize_body,
        grid=(NQPAD // BQ,),
        in_specs=[pl.BlockSpec((NC, BQ, SCW), lambda i: (0, i, 0))],
        out_specs=pl.BlockSpec((BQ, ODIM), lambda i: (i, 0)),
        out_shape=jax.ShapeDtypeStruct((NQPAD, ODIM), jnp.float32),
    )(acc2)
    return out[:NQ]
